# baseline (speedup 1.0000x reference)
import jax
import jax.numpy as jnp
from jax.experimental import pallas as pl
from jax.experimental.pallas import tpu as pltpu


BLK = 512
NBUF = 5


def _outer(x_hbm, w_ref, b_ref, out_hbm, tail_x, tail_out, tail_sem, out_sem):
    R, K = x_hbm.shape
    C = w_ref.shape[0]
    steps = R // BLK
    tail_base = steps * BLK
    tail_rows = R - tail_base

    tail_copy = pltpu.make_async_copy(
        x_hbm.at[pl.ds(tail_base, tail_rows), :], tail_x, tail_sem)
    tail_copy.start()

    def inner(idx, x_tile, out_tile):
        out_tile[...] = (
            jax.lax.dot_general(
                w_ref[...], x_tile[...],
                dimension_numbers=(((1,), (1,)), ((), ())),
                preferred_element_type=jnp.float32,
            )
            + b_ref[...]
        )

    pltpu.emit_pipeline(
        inner,
        grid=(steps,),
        in_specs=[
            pl.BlockSpec((BLK, K), lambda i: (i, 0),
                         pipeline_mode=pl.Buffered(buffer_count=NBUF)),
        ],
        out_specs=[
            pl.BlockSpec((C, BLK), lambda i: (0, i)),
        ],
        _explicit_indices=True,
    )(x_hbm, out_hbm)

    tail_copy.wait()
    tail_out[...] = (
        jax.lax.dot_general(
            w_ref[...], tail_x[...],
            dimension_numbers=(((1,), (1,)), ((), ())),
            preferred_element_type=jnp.float32,
        )
        + b_ref[...]
    )
    tail_store = pltpu.make_async_copy(
        tail_out, out_hbm.at[:, pl.ds(tail_base, tail_rows)], out_sem)
    tail_store.start()
    tail_store.wait()


def kernel(x, W, b):
    R, K = x.shape
    C = W.shape[0]
    b2 = b.reshape(C, 1)
    tail_rows = R - (R // BLK) * BLK
    out_t = pl.pallas_call(
        _outer,
        in_specs=[
            pl.BlockSpec(memory_space=pl.ANY),
            pl.BlockSpec((C, K), lambda: (0, 0)),
            pl.BlockSpec((C, 1), lambda: (0, 0)),
        ],
        out_specs=pl.BlockSpec(memory_space=pl.ANY),
        out_shape=jax.ShapeDtypeStruct((C, R), jnp.float32),
        scratch_shapes=[
            pltpu.VMEM((tail_rows, K), jnp.float32),
            pltpu.VMEM((C, tail_rows), jnp.float32),
            pltpu.SemaphoreType.DMA,
            pltpu.SemaphoreType.DMA,
        ],
    )(x, W, b2)
    return out_t.T


# speedup vs baseline: 1.0207x; 1.0207x over previous
import jax
import jax.numpy as jnp
from jax.experimental import pallas as pl
from jax.experimental.pallas import tpu as pltpu


BLK = 512
NBUF = 4


def _outer(x_hbm, w_ref, b_ref, out_hbm, tail_x, tail_out, tail_sem, out_sem):
    R, K = x_hbm.shape
    C = w_ref.shape[0]
    steps = R // BLK
    tail_base = steps * BLK
    tail_rows = R - tail_base

    tail_copy = pltpu.make_async_copy(
        x_hbm.at[pl.ds(tail_base, tail_rows), :], tail_x, tail_sem)
    tail_copy.start()

    def inner(idx, x_tile, out_tile):
        out_tile[...] = (
            jax.lax.dot_general(
                w_ref[...], x_tile[...],
                dimension_numbers=(((1,), (1,)), ((), ())),
                preferred_element_type=jnp.float32,
            )
            + b_ref[...]
        )

    pltpu.emit_pipeline(
        inner,
        grid=(steps,),
        in_specs=[
            pl.BlockSpec((BLK, K), lambda i: (i, 0),
                         pipeline_mode=pl.Buffered(buffer_count=NBUF)),
        ],
        out_specs=[
            pl.BlockSpec((C, BLK), lambda i: (0, i)),
        ],
        _explicit_indices=True,
    )(x_hbm, out_hbm)

    tail_copy.wait()
    tail_out[...] = (
        jax.lax.dot_general(
            w_ref[...], tail_x[...],
            dimension_numbers=(((1,), (1,)), ((), ())),
            preferred_element_type=jnp.float32,
        )
        + b_ref[...]
    )
    tail_store = pltpu.make_async_copy(
        tail_out, out_hbm.at[:, pl.ds(tail_base, tail_rows)], out_sem)
    tail_store.start()
    tail_store.wait()


def kernel(x, W, b):
    R, K = x.shape
    C = W.shape[0]
    b2 = b.reshape(C, 1)
    tail_rows = R - (R // BLK) * BLK
    out_t = pl.pallas_call(
        _outer,
        in_specs=[
            pl.BlockSpec(memory_space=pl.ANY),
            pl.BlockSpec((C, K), lambda: (0, 0)),
            pl.BlockSpec((C, 1), lambda: (0, 0)),
        ],
        out_specs=pl.BlockSpec(memory_space=pl.ANY),
        out_shape=jax.ShapeDtypeStruct((C, R), jnp.float32),
        scratch_shapes=[
            pltpu.VMEM((tail_rows, K), jnp.float32),
            pltpu.VMEM((C, tail_rows), jnp.float32),
            pltpu.SemaphoreType.DMA,
            pltpu.SemaphoreType.DMA,
        ],
    )(x, W, b2)
    return out_t.T
